# baseline (speedup 1.0000x reference)
import jax
import jax.numpy as jnp
from jax import lax
from jax.experimental import pallas as pl
from jax.experimental.pallas import tpu as pltpu
from jax.experimental.pallas import tpu_sc as plsc

N = 10000
DIN = 128
H = 16

NC = 2
NS = 16
L = 16
NW = NC * NS
K = 128
NWIN = 80
T_EDGES = K * NWIN
E_CAP = NW * T_EDGES
NPAD = 10240
RPT = NPAD // NS

def _deg_body(dst_hbm, w_hbm, zeros_hbm, out_hbm, dst_v, w_v, dsem, deg_sh):
    cid = lax.axis_index("c")
    sid = lax.axis_index("s")
    wid = sid * NC + cid
    pltpu.sync_copy(dst_hbm.at[wid], dst_v)
    pltpu.sync_copy(w_hbm.at[wid], w_v)
    pltpu.sync_copy(zeros_hbm.at[pl.ds(sid * RPT, RPT)],
                    deg_sh.at[pl.ds(sid * RPT, RPT)])
    plsc.subcore_barrier()

    def body(g, carry):
        pltpu.async_copy(w_v.at[g], deg_sh.at[dst_v.at[g]], dsem, add=True)

        @pl.when(g >= 8)
        def _():
            pltpu.make_async_copy(w_v.at[g], deg_sh.at[dst_v.at[g]],
                                  dsem).wait()
        return carry

    lax.fori_loop(0, NWIN, body, 0)
    for g in range(8):
        pltpu.make_async_copy(w_v.at[g], deg_sh.at[dst_v.at[g]], dsem).wait()
    plsc.subcore_barrier()
    pltpu.sync_copy(deg_sh.at[pl.ds(sid * RPT, RPT)],
                    out_hbm.at[cid, pl.ds(sid * RPT, RPT)])


import functools


@functools.cache
def _sc_kernels():
    mesh = plsc.VectorSubcoreMesh(
        core_axis_name="c", subcore_axis_name="s",
        num_cores=NC, num_subcores=NS,
    )
    deg_kernel = pl.kernel(
        _deg_body,
        out_type=jax.ShapeDtypeStruct((NC, NPAD), jnp.float32),
        mesh=mesh,
        scratch_types=[
            pltpu.VMEM((NWIN, K), jnp.int32),
            pltpu.VMEM((NWIN, K), jnp.float32),
            pltpu.SemaphoreType.DMA,
            pltpu.VMEM_SHARED((NPAD,), jnp.float32),
        ],
        compiler_params=pltpu.CompilerParams(use_tc_tiling_on_sc=False),
    )
    prop_scratch = [
        pltpu.VMEM((NWIN + NBUF, K), jnp.int32),
        pltpu.VMEM((NWIN, K), jnp.int32),
        pltpu.VMEM((NWIN, K), jnp.float32),
        pltpu.VMEM((NBUF, K, H), jnp.float32),
        pltpu.SemaphoreType.DMA((NBUF,)),
        pltpu.SemaphoreType.DMA((NBUF,)),
    ]
    prop_kernel = pl.kernel(
        _prop_body,
        out_type=jax.ShapeDtypeStruct((NC, NPAD, H), jnp.float32),
        mesh=mesh,
        scratch_types=prop_scratch + [
            pltpu.VMEM_SHARED((NPAD, H), jnp.float32),
            pltpu.VMEM_SHARED((NPAD, H), jnp.float32),
        ],
        compiler_params=pltpu.CompilerParams(use_tc_tiling_on_sc=False),
    )
    prop_mm_kernel = pl.kernel(
        _prop_mm_body,
        out_type=(jax.ShapeDtypeStruct((NC, NPAD, H), jnp.float32),
                  jax.ShapeDtypeStruct((NPAD, H), jnp.float32),
                  jax.ShapeDtypeStruct((NPAD, H), jnp.float32)),
        mesh=mesh,
        scratch_types=prop_scratch + [
            pltpu.VMEM((RPT, H), jnp.float32),
            pltpu.VMEM((RPT, H), jnp.float32),
            pltpu.VMEM((RPT, H), jnp.float32),
            pltpu.VMEM((RPT, H), jnp.float32),
            pltpu.VMEM((RPT,), jnp.float32),
            pltpu.VMEM((L,), jnp.float32),
            pltpu.VMEM((L, H), jnp.float32),
            pltpu.VMEM((RPT, H), jnp.float32),
            pltpu.VMEM((RPT, H), jnp.float32),
            pltpu.VMEM_SHARED((NPAD, H), jnp.float32),
            pltpu.VMEM_SHARED((NPAD, H), jnp.float32),
        ],
        compiler_params=pltpu.CompilerParams(use_tc_tiling_on_sc=False),
    )
    prop_epi_kernel = pl.kernel(
        _prop_epi_body,
        out_type=(jax.ShapeDtypeStruct((NC, NPAD, H), jnp.float32),
                  jax.ShapeDtypeStruct((NPAD, H), jnp.float32)),
        mesh=mesh,
        scratch_types=prop_scratch + [
            pltpu.VMEM((RPT, H), jnp.float32),
            pltpu.VMEM((RPT, H), jnp.float32),
            pltpu.VMEM((RPT, H), jnp.float32),
            pltpu.VMEM((RPT, H), jnp.float32),
            pltpu.VMEM((RPT,), jnp.float32),
            pltpu.VMEM((L,), jnp.float32),
            pltpu.VMEM((RPT, H), jnp.float32),
            pltpu.VMEM_SHARED((NPAD, H), jnp.float32),
            pltpu.VMEM_SHARED((NPAD, H), jnp.float32),
        ],
        compiler_params=pltpu.CompilerParams(use_tc_tiling_on_sc=False),
    )
    return deg_kernel, prop_kernel, prop_mm_kernel, prop_epi_kernel


NBUF = 4


LASTR = N - (NS - 1) * RPT


def _stage_edges_and_zero(src_hbm, dst_hbm, w_hbm, zeros_hbm,
                          src_v, dst_v, w_v, acc_sh, wid, sid):
    pltpu.sync_copy(src_hbm.at[wid], src_v)
    pltpu.sync_copy(dst_hbm.at[wid], dst_v)
    pltpu.sync_copy(w_hbm.at[wid], w_v)
    pltpu.sync_copy(zeros_hbm.at[pl.ds(sid * RPT, RPT)],
                    acc_sh.at[pl.ds(sid * RPT, RPT)])


def _propagate(src_v, dst_v, w_v, rows_v, gsems, ssems, tbl_sh, acc_sh):
    for b in range(NBUF):
        pltpu.async_copy(tbl_sh.at[src_v.at[b]], rows_v.at[b], gsems.at[b])

    def scale(g, b):
        rows = rows_v.at[b]
        for j16 in range(K // L):
            w16 = w_v[g, j16 * L:(j16 + 1) * L]
            for j in range(L):
                e = j16 * L + j
                ws = jnp.take_along_axis(
                    w16, jnp.full((L,), j, jnp.int32), axis=0)
                rows[e, :] = rows[e, :] * ws

    def body(g4, carry):
        for b in range(NBUF):
            g = g4 * NBUF + b
            pltpu.make_async_copy(
                tbl_sh.at[src_v.at[g]], rows_v.at[b], gsems.at[b]).wait()
            scale(g, b)
            pltpu.async_copy(rows_v.at[b], acc_sh.at[dst_v.at[g]],
                             ssems.at[b], add=True)
        for b in range(NBUF):
            g = g4 * NBUF + b
            pltpu.make_async_copy(rows_v.at[b], acc_sh.at[dst_v.at[g]],
                                  ssems.at[b]).wait()
            pltpu.async_copy(tbl_sh.at[src_v.at[g + NBUF]], rows_v.at[b],
                             gsems.at[b])
        return carry

    lax.fori_loop(0, NWIN // NBUF, body, 0)
    for b in range(NBUF):
        pltpu.make_async_copy(
            tbl_sh.at[src_v.at[b]], rows_v.at[b], gsems.at[b]).wait()


def _prop_body(t_hbm, src_hbm, dst_hbm, w_hbm, zeros_hbm, out_hbm,
               src_v, dst_v, w_v, rows_v, gsems, ssems, tbl_sh, acc_sh):
    cid = lax.axis_index("c")
    sid = lax.axis_index("s")
    wid = sid * NC + cid
    _stage_edges_and_zero(src_hbm, dst_hbm, w_hbm, zeros_hbm,
                          src_v, dst_v, w_v, acc_sh, wid, sid)

    @pl.when(sid < NS - 1)
    def _():
        pltpu.sync_copy(t_hbm.at[pl.ds(sid * RPT, RPT)],
                        tbl_sh.at[pl.ds(sid * RPT, RPT)])

    @pl.when(sid == NS - 1)
    def _():
        pltpu.sync_copy(t_hbm.at[pl.ds((NS - 1) * RPT, LASTR)],
                        tbl_sh.at[pl.ds((NS - 1) * RPT, LASTR)])

    plsc.subcore_barrier()
    _propagate(src_v, dst_v, w_v, rows_v, gsems, ssems, tbl_sh, acc_sh)
    plsc.subcore_barrier()
    pltpu.sync_copy(acc_sh.at[pl.ds(sid * RPT, RPT)],
                    out_hbm.at[cid, pl.ds(sid * RPT, RPT)])


def _prop_mm_body(accp_hbm, tp_hbm, res_hbm, dis_hbm, bp_hbm, wn_hbm,
                  src_hbm, dst_hbm, w_hbm, zeros_hbm,
                  out_hbm, h_hbm, tn_hbm,
                  src_v, dst_v, w_v, rows_v, gsems, ssems,
                  a0_v, a1_v, tp_v, res_v, dis_v, bp_v, wn_v, h_v, tn_v,
                  tbl_sh, acc_sh):
    cid = lax.axis_index("c")
    sid = lax.axis_index("s")
    wid = sid * NC + cid
    _stage_edges_and_zero(src_hbm, dst_hbm, w_hbm, zeros_hbm,
                          src_v, dst_v, w_v, acc_sh, wid, sid)

    base = sid * RPT

    @pl.when(sid < NS - 1)
    def _():
        pltpu.sync_copy(accp_hbm.at[0, pl.ds(base, RPT)], a0_v)
        pltpu.sync_copy(accp_hbm.at[1, pl.ds(base, RPT)], a1_v)
        pltpu.sync_copy(tp_hbm.at[pl.ds(base, RPT)], tp_v)
        pltpu.sync_copy(res_hbm.at[pl.ds(base, RPT)], res_v)
        pltpu.sync_copy(dis_hbm.at[pl.ds(base, RPT)], dis_v)

    @pl.when(sid == NS - 1)
    def _():
        pltpu.sync_copy(accp_hbm.at[0, pl.ds(base, LASTR)],
                        a0_v.at[pl.ds(0, LASTR)])
        pltpu.sync_copy(accp_hbm.at[1, pl.ds(base, LASTR)],
                        a1_v.at[pl.ds(0, LASTR)])
        pltpu.sync_copy(tp_hbm.at[pl.ds(base, LASTR)],
                        tp_v.at[pl.ds(0, LASTR)])
        pltpu.sync_copy(res_hbm.at[pl.ds(base, LASTR)],
                        res_v.at[pl.ds(0, LASTR)])
        pltpu.sync_copy(dis_hbm.at[pl.ds(base, LASTR)],
                        dis_v.at[pl.ds(0, LASTR)])

    pltpu.sync_copy(bp_hbm, bp_v)
    pltpu.sync_copy(wn_hbm, wn_v)
    bpv = bp_v[0:L]
    wrows = [wn_v[j, :] for j in range(L)]
    ngroups = jnp.where(sid == NS - 1, LASTR // L, RPT // L)

    def build(i, carry):
        dis16 = dis_v[pl.ds(i * L, L)]
        for j in range(L):
            r = i * L + j
            ds = jnp.take_along_axis(dis16, jnp.full((L,), j, jnp.int32),
                                     axis=0)
            conv = ds * (a0_v[r, :] + a1_v[r, :] + tp_v[r, :]) + bpv
            h = jnp.maximum(conv + res_v[r, :], 0.0)
            acc = jnp.take_along_axis(h, jnp.full((L,), 0, jnp.int32),
                                      axis=0) * wrows[0]
            for f in range(1, L):
                acc = acc + jnp.take_along_axis(
                    h, jnp.full((L,), f, jnp.int32), axis=0) * wrows[f]
            h_v[r, :] = h
            tn_v[r, :] = ds * acc
        return carry

    lax.fori_loop(0, ngroups, build, 0)

    @pl.when(sid < NS - 1)
    def _():
        pltpu.sync_copy(tn_v, tbl_sh.at[pl.ds(base, RPT)])

    @pl.when(sid == NS - 1)
    def _():
        pltpu.sync_copy(tn_v.at[pl.ds(0, LASTR)],
                        tbl_sh.at[pl.ds(base, LASTR)])

    @pl.when(cid == 0)
    def _():
        pltpu.sync_copy(h_v, h_hbm.at[pl.ds(base, RPT)])
        pltpu.sync_copy(tn_v, tn_hbm.at[pl.ds(base, RPT)])

    plsc.subcore_barrier()
    _propagate(src_v, dst_v, w_v, rows_v, gsems, ssems, tbl_sh, acc_sh)
    plsc.subcore_barrier()
    pltpu.sync_copy(acc_sh.at[pl.ds(sid * RPT, RPT)],
                    out_hbm.at[cid, pl.ds(sid * RPT, RPT)])


def _prop_epi_body(acc3_hbm, t3_hbm, h2_hbm, dis_hbm, b3_hbm,
                   src_hbm, dst_hbm, w_hbm, zeros_hbm,
                   out_hbm, t4_hbm,
                   src_v, dst_v, w_v, rows_v, gsems, ssems,
                   a0_v, a1_v, t3_v, h2_v, dis_v, b3_v, t4_v,
                   tbl_sh, acc_sh):
    cid = lax.axis_index("c")
    sid = lax.axis_index("s")
    wid = sid * NC + cid
    _stage_edges_and_zero(src_hbm, dst_hbm, w_hbm, zeros_hbm,
                          src_v, dst_v, w_v, acc_sh, wid, sid)

    base = sid * RPT

    @pl.when(sid < NS - 1)
    def _():
        pltpu.sync_copy(acc3_hbm.at[0, pl.ds(base, RPT)], a0_v)
        pltpu.sync_copy(acc3_hbm.at[1, pl.ds(base, RPT)], a1_v)
        pltpu.sync_copy(t3_hbm.at[pl.ds(base, RPT)], t3_v)
        pltpu.sync_copy(h2_hbm.at[pl.ds(base, RPT)], h2_v)
        pltpu.sync_copy(dis_hbm.at[pl.ds(base, RPT)], dis_v)

    @pl.when(sid == NS - 1)
    def _():
        pltpu.sync_copy(acc3_hbm.at[0, pl.ds(base, LASTR)],
                        a0_v.at[pl.ds(0, LASTR)])
        pltpu.sync_copy(acc3_hbm.at[1, pl.ds(base, LASTR)],
                        a1_v.at[pl.ds(0, LASTR)])
        pltpu.sync_copy(t3_hbm.at[pl.ds(base, LASTR)],
                        t3_v.at[pl.ds(0, LASTR)])
        pltpu.sync_copy(h2_hbm.at[pl.ds(base, LASTR)],
                        h2_v.at[pl.ds(0, LASTR)])
        pltpu.sync_copy(dis_hbm.at[pl.ds(base, LASTR)],
                        dis_v.at[pl.ds(0, LASTR)])

    pltpu.sync_copy(b3_hbm, b3_v)
    b3v = b3_v[0:L]
    ngroups = jnp.where(sid == NS - 1, LASTR // L, RPT // L)

    def build(i, carry):
        dis16 = dis_v[pl.ds(i * L, L)]
        for j in range(L):
            r = i * L + j
            ds = jnp.take_along_axis(dis16, jnp.full((L,), j, jnp.int32),
                                     axis=0)
            conv = ds * (a0_v[r, :] + a1_v[r, :] + t3_v[r, :]) + b3v
            h3 = jnp.maximum(conv + h2_v[r, :], 0.0)
            t4_v[r, :] = ds * h3
        return carry

    lax.fori_loop(0, ngroups, build, 0)

    @pl.when(sid < NS - 1)
    def _():
        pltpu.sync_copy(t4_v, tbl_sh.at[pl.ds(base, RPT)])

    @pl.when(sid == NS - 1)
    def _():
        pltpu.sync_copy(t4_v.at[pl.ds(0, LASTR)],
                        tbl_sh.at[pl.ds(base, LASTR)])

    @pl.when(cid == 0)
    def _():
        pltpu.sync_copy(t4_v, t4_hbm.at[pl.ds(base, RPT)])

    plsc.subcore_barrier()
    _propagate(src_v, dst_v, w_v, rows_v, gsems, ssems, tbl_sh, acc_sh)
    plsc.subcore_barrier()
    pltpu.sync_copy(acc_sh.at[pl.ds(sid * RPT, RPT)],
                    out_hbm.at[cid, pl.ds(sid * RPT, RPT)])


BN = 1000


def _tc_a_body(deg_ref, x_ref, W1_ref, Wres_ref, bres_ref,
               t1_ref, xres_ref, dis_ref):
    deg = deg_ref[0] + deg_ref[1] + 1.0
    dis = lax.rsqrt(deg)
    xw = jnp.dot(x_ref[...], W1_ref[...], preferred_element_type=jnp.float32,
                precision=jax.lax.Precision.HIGHEST)
    t1_ref[...] = xw * dis
    xres_ref[...] = (
        jnp.dot(x_ref[...], Wres_ref[...], preferred_element_type=jnp.float32,
                precision=jax.lax.Precision.HIGHEST)
        + bres_ref[...]
    )
    dis_ref[...] = dis


_tc_a = pl.pallas_call(
    _tc_a_body,
    grid=(N // BN,),
    in_specs=[
        pl.BlockSpec((NC, BN, 1), lambda i: (0, i, 0)),
        pl.BlockSpec((BN, DIN), lambda i: (i, 0)),
        pl.BlockSpec((DIN, H), lambda i: (0, 0)),
        pl.BlockSpec((DIN, H), lambda i: (0, 0)),
        pl.BlockSpec((1, H), lambda i: (0, 0)),
    ],
    out_specs=[
        pl.BlockSpec((BN, H), lambda i: (i, 0)),
        pl.BlockSpec((BN, H), lambda i: (i, 0)),
        pl.BlockSpec((BN, 1), lambda i: (i, 0)),
    ],
    out_shape=[
        jax.ShapeDtypeStruct((N, H), jnp.float32),
        jax.ShapeDtypeStruct((N, H), jnp.float32),
        jax.ShapeDtypeStruct((N, 1), jnp.float32),
    ],
)


def _tc_b_body(acc_ref, t_ref, res_ref, dis_ref, b_ref, Wn_ref,
               h_ref, tn_ref):
    conv = (acc_ref[0] + acc_ref[1] + t_ref[...]) * dis_ref[...] + b_ref[...]
    h = jnp.maximum(conv + res_ref[...], 0.0)
    h_ref[...] = h
    tn_ref[...] = (
        jnp.dot(h, Wn_ref[...], preferred_element_type=jnp.float32,
                precision=jax.lax.Precision.HIGHEST)
        * dis_ref[...]
    )


_tc_b = pl.pallas_call(
    _tc_b_body,
    grid=(N // BN,),
    in_specs=[
        pl.BlockSpec((NC, BN, H), lambda i: (0, i, 0)),
        pl.BlockSpec((BN, H), lambda i: (i, 0)),
        pl.BlockSpec((BN, H), lambda i: (i, 0)),
        pl.BlockSpec((BN, 1), lambda i: (i, 0)),
        pl.BlockSpec((1, H), lambda i: (0, 0)),
        pl.BlockSpec((H, H), lambda i: (0, 0)),
    ],
    out_specs=[
        pl.BlockSpec((BN, H), lambda i: (i, 0)),
        pl.BlockSpec((BN, H), lambda i: (i, 0)),
    ],
    out_shape=[
        jax.ShapeDtypeStruct((N, H), jnp.float32),
        jax.ShapeDtypeStruct((N, H), jnp.float32),
    ],
)


def _tc_b3_body(acc_ref, t_ref, res_ref, dis_ref, b_ref, t4_ref):
    conv = (acc_ref[0] + acc_ref[1] + t_ref[...]) * dis_ref[...] + b_ref[...]
    h = jnp.maximum(conv + res_ref[...], 0.0)
    t4_ref[...] = h * dis_ref[...]


_tc_b3 = pl.pallas_call(
    _tc_b3_body,
    grid=(N // BN,),
    in_specs=[
        pl.BlockSpec((NC, BN, H), lambda i: (0, i, 0)),
        pl.BlockSpec((BN, H), lambda i: (i, 0)),
        pl.BlockSpec((BN, H), lambda i: (i, 0)),
        pl.BlockSpec((BN, 1), lambda i: (i, 0)),
        pl.BlockSpec((1, H), lambda i: (0, 0)),
    ],
    out_specs=[pl.BlockSpec((BN, H), lambda i: (i, 0))],
    out_shape=[jax.ShapeDtypeStruct((N, H), jnp.float32)],
)


def _tc_c_body(acc_ref, t_ref, dis_ref, W4_ref, b4_ref, out_ref):
    z = (acc_ref[0] + acc_ref[1] + t_ref[...]) * dis_ref[...]
    out_ref[...] = (
        jnp.dot(z, W4_ref[...], preferred_element_type=jnp.float32,
                precision=jax.lax.Precision.HIGHEST)
        + b4_ref[...]
    )


_tc_c = pl.pallas_call(
    _tc_c_body,
    grid=(N // BN,),
    in_specs=[
        pl.BlockSpec((NC, BN, H), lambda i: (0, i, 0)),
        pl.BlockSpec((BN, H), lambda i: (i, 0)),
        pl.BlockSpec((BN, 1), lambda i: (i, 0)),
        pl.BlockSpec((H, 1), lambda i: (0, 0)),
        pl.BlockSpec((1, 1), lambda i: (0, 0)),
    ],
    out_specs=[pl.BlockSpec((BN, 1), lambda i: (i, 0))],
    out_shape=[jax.ShapeDtypeStruct((N, 1), jnp.float32)],
)


def kernel(x, edge_index, edge_weight, W1, b1, W2, b2, W3, b3, W4, b4,
           Wres, bres):
    src = edge_index[0].astype(jnp.int32)
    dst = edge_index[1].astype(jnp.int32)
    w = edge_weight.astype(jnp.float32)
    e_in = src.shape[0]
    pad = E_CAP - e_in

    spread = (jnp.arange(pad, dtype=jnp.int32) * 97) % N
    src_p = jnp.concatenate([src, spread]).reshape(NW, NWIN, K)
    dummy = jnp.broadcast_to(
        ((jnp.arange(NBUF * K, dtype=jnp.int32) * 53) % N).reshape(1, NBUF, K),
        (NW, NBUF, K),
    )
    src3 = jnp.concatenate([src_p, dummy], axis=1)
    dst3 = jnp.concatenate([dst, spread]).reshape(NW, NWIN, K)
    w3 = jnp.concatenate([w, jnp.zeros((pad,), jnp.float32)]).reshape(
        NW, NWIN, K)
    zeros_n = jnp.zeros((NPAD,), jnp.float32)
    zeros_nh = jnp.zeros((NPAD, H), jnp.float32)

    _deg_kernel, _prop_kernel, _prop_mm_kernel, _prop_epi_kernel = \
        _sc_kernels()
    deg_parts = _deg_kernel(dst3, w3, zeros_n)
    t1, xres, dis = _tc_a(deg_parts.reshape(NC, NPAD, 1), x, W1, Wres,
                          bres.reshape(1, H))
    dis1 = dis.reshape(N)
    acc1 = _prop_kernel(t1, src3, dst3, w3, zeros_nh)
    acc2, h1, t2 = _prop_mm_kernel(acc1, t1, xres, dis1, b1, W2,
                                   src3, dst3, w3, zeros_nh)
    acc3, h2, t3 = _prop_mm_kernel(acc2, t2[:N], h1[:N], dis1, b2, W3,
                                   src3, dst3, w3, zeros_nh)
    acc4, t4 = _prop_epi_kernel(acc3, t3[:N], h2[:N], dis1, b3,
                                src3, dst3, w3, zeros_nh)
    (out,) = _tc_c(acc4, t4[:N], dis, W4, b4.reshape(1, 1))
    return out


# speedup vs baseline: 1.1650x; 1.1650x over previous
import jax
import jax.numpy as jnp
from jax import lax
from jax.experimental import pallas as pl
from jax.experimental.pallas import tpu as pltpu
from jax.experimental.pallas import tpu_sc as plsc

N = 10000
DIN = 128
H = 16

NC = 2
NS = 16
L = 16
NW = NC * NS
K = 128
NWIN = 80
T_EDGES = K * NWIN
E_CAP = NW * T_EDGES
NPAD = 10240
RPT = NPAD // NS

def _deg_body(dst_hbm, w_hbm, zeros_hbm, out_hbm, dst_v, w_v, dsem, deg_sh):
    cid = lax.axis_index("c")
    sid = lax.axis_index("s")
    wid = sid * NC + cid
    pltpu.sync_copy(dst_hbm.at[wid], dst_v)
    pltpu.sync_copy(w_hbm.at[wid], w_v)
    pltpu.sync_copy(zeros_hbm.at[pl.ds(sid * RPT, RPT)],
                    deg_sh.at[pl.ds(sid * RPT, RPT)])
    plsc.subcore_barrier()

    def body(g, carry):
        pltpu.async_copy(w_v.at[g], deg_sh.at[dst_v.at[g]], dsem, add=True)

        @pl.when(g >= 8)
        def _():
            pltpu.make_async_copy(w_v.at[g], deg_sh.at[dst_v.at[g]],
                                  dsem).wait()
        return carry

    lax.fori_loop(0, NWIN, body, 0)
    for g in range(8):
        pltpu.make_async_copy(w_v.at[g], deg_sh.at[dst_v.at[g]], dsem).wait()
    plsc.subcore_barrier()
    pltpu.sync_copy(deg_sh.at[pl.ds(sid * RPT, RPT)],
                    out_hbm.at[cid, pl.ds(sid * RPT, RPT)])


import functools


@functools.cache
def _sc_kernels():
    mesh = plsc.VectorSubcoreMesh(
        core_axis_name="c", subcore_axis_name="s",
        num_cores=NC, num_subcores=NS,
    )
    deg_kernel = pl.kernel(
        _deg_body,
        out_type=jax.ShapeDtypeStruct((NC, NPAD), jnp.float32),
        mesh=mesh,
        scratch_types=[
            pltpu.VMEM((NWIN, K), jnp.int32),
            pltpu.VMEM((NWIN, K), jnp.float32),
            pltpu.SemaphoreType.DMA,
            pltpu.VMEM_SHARED((NPAD,), jnp.float32),
        ],
        compiler_params=pltpu.CompilerParams(use_tc_tiling_on_sc=False),
    )
    prop_scratch = [
        pltpu.VMEM((NWIN + NBUF, K), jnp.int32),
        pltpu.VMEM((NWIN, K), jnp.int32),
        pltpu.VMEM((NWIN, K), jnp.float32),
        pltpu.VMEM((NBUF, K, H), jnp.float32),
        pltpu.SemaphoreType.DMA((NBUF,)),
        pltpu.SemaphoreType.DMA((NBUF,)),
    ]
    prop_kernel = pl.kernel(
        _prop_body,
        out_type=jax.ShapeDtypeStruct((NC, NPAD, H), jnp.float32),
        mesh=mesh,
        scratch_types=prop_scratch + [
            pltpu.VMEM_SHARED((NPAD, H), jnp.float32),
            pltpu.VMEM_SHARED((NPAD, H), jnp.float32),
        ],
        compiler_params=pltpu.CompilerParams(use_tc_tiling_on_sc=False),
    )
    prop_epi_kernel = pl.kernel(
        _prop_epi_body,
        out_type=(jax.ShapeDtypeStruct((NC, NPAD, H), jnp.float32),
                  jax.ShapeDtypeStruct((NPAD, H), jnp.float32)),
        mesh=mesh,
        scratch_types=prop_scratch + [
            pltpu.VMEM((RPT, H), jnp.float32),
            pltpu.VMEM((RPT, H), jnp.float32),
            pltpu.VMEM((RPT, H), jnp.float32),
            pltpu.VMEM((RPT, H), jnp.float32),
            pltpu.VMEM((RPT,), jnp.float32),
            pltpu.VMEM((L,), jnp.float32),
            pltpu.VMEM((RPT, H), jnp.float32),
            pltpu.VMEM_SHARED((NPAD, H), jnp.float32),
            pltpu.VMEM_SHARED((NPAD, H), jnp.float32),
        ],
        compiler_params=pltpu.CompilerParams(use_tc_tiling_on_sc=False),
    )
    return deg_kernel, prop_kernel, prop_epi_kernel


NBUF = 4


LASTR = N - (NS - 1) * RPT


def _stage_edges_and_zero(src_hbm, dst_hbm, w_hbm, zeros_hbm,
                          src_v, dst_v, w_v, acc_sh, wid, sid, gsems):
    return [
        pltpu.async_copy(src_hbm.at[wid], src_v, gsems.at[0]),
        pltpu.async_copy(dst_hbm.at[wid], dst_v, gsems.at[1]),
        pltpu.async_copy(w_hbm.at[wid], w_v, gsems.at[2]),
        pltpu.async_copy(zeros_hbm.at[pl.ds(sid * RPT, RPT)],
                         acc_sh.at[pl.ds(sid * RPT, RPT)], gsems.at[3]),
    ]


def _propagate(src_v, dst_v, w_v, rows_v, gsems, ssems, tbl_sh, acc_sh):
    for b in range(NBUF):
        pltpu.async_copy(tbl_sh.at[src_v.at[b]], rows_v.at[b], gsems.at[b])

    def scale(g, b):
        rows = rows_v.at[b]
        for j16 in range(K // L):
            w16 = w_v[g, j16 * L:(j16 + 1) * L]
            for j in range(L):
                e = j16 * L + j
                ws = jnp.take_along_axis(
                    w16, jnp.full((L,), j, jnp.int32), axis=0)
                rows[e, :] = rows[e, :] * ws

    def body(g4, carry):
        for b in range(NBUF):
            g = g4 * NBUF + b
            pltpu.make_async_copy(
                tbl_sh.at[src_v.at[g]], rows_v.at[b], gsems.at[b]).wait()
            scale(g, b)
            pltpu.async_copy(rows_v.at[b], acc_sh.at[dst_v.at[g]],
                             ssems.at[b], add=True)
        for b in range(NBUF):
            g = g4 * NBUF + b
            pltpu.make_async_copy(rows_v.at[b], acc_sh.at[dst_v.at[g]],
                                  ssems.at[b]).wait()
            pltpu.async_copy(tbl_sh.at[src_v.at[g + NBUF]], rows_v.at[b],
                             gsems.at[b])
        return carry

    lax.fori_loop(0, NWIN // NBUF, body, 0)
    for b in range(NBUF):
        pltpu.make_async_copy(
            tbl_sh.at[src_v.at[b]], rows_v.at[b], gsems.at[b]).wait()


def _prop_body(t_hbm, src_hbm, dst_hbm, w_hbm, zeros_hbm, out_hbm,
               src_v, dst_v, w_v, rows_v, gsems, ssems, tbl_sh, acc_sh):
    cid = lax.axis_index("c")
    sid = lax.axis_index("s")
    wid = sid * NC + cid
    stage = _stage_edges_and_zero(src_hbm, dst_hbm, w_hbm, zeros_hbm,
                                  src_v, dst_v, w_v, acc_sh, wid, sid, gsems)

    @pl.when(sid < NS - 1)
    def _():
        pltpu.sync_copy(t_hbm.at[pl.ds(sid * RPT, RPT)],
                        tbl_sh.at[pl.ds(sid * RPT, RPT)])

    @pl.when(sid == NS - 1)
    def _():
        pltpu.sync_copy(t_hbm.at[pl.ds((NS - 1) * RPT, LASTR)],
                        tbl_sh.at[pl.ds((NS - 1) * RPT, LASTR)])

    for d in stage:
        d.wait()
    plsc.subcore_barrier()
    _propagate(src_v, dst_v, w_v, rows_v, gsems, ssems, tbl_sh, acc_sh)
    plsc.subcore_barrier()
    pltpu.sync_copy(acc_sh.at[pl.ds(sid * RPT, RPT)],
                    out_hbm.at[cid, pl.ds(sid * RPT, RPT)])


def _prop_epi_body(acc3_hbm, t3_hbm, h2_hbm, dis_hbm, b3_hbm,
                   src_hbm, dst_hbm, w_hbm, zeros_hbm,
                   out_hbm, t4_hbm,
                   src_v, dst_v, w_v, rows_v, gsems, ssems,
                   a0_v, a1_v, t3_v, h2_v, dis_v, b3_v, t4_v,
                   tbl_sh, acc_sh):
    cid = lax.axis_index("c")
    sid = lax.axis_index("s")
    wid = sid * NC + cid
    stage = _stage_edges_and_zero(src_hbm, dst_hbm, w_hbm, zeros_hbm,
                                  src_v, dst_v, w_v, acc_sh, wid, sid, gsems)

    base = sid * RPT

    @pl.when(sid < NS - 1)
    def _():
        ds_ = [
            pltpu.async_copy(acc3_hbm.at[0, pl.ds(base, RPT)], a0_v,
                             ssems.at[0]),
            pltpu.async_copy(acc3_hbm.at[1, pl.ds(base, RPT)], a1_v,
                             ssems.at[1]),
            pltpu.async_copy(t3_hbm.at[pl.ds(base, RPT)], t3_v,
                             ssems.at[2]),
            pltpu.async_copy(h2_hbm.at[pl.ds(base, RPT)], h2_v,
                             ssems.at[3]),
        ]
        pltpu.sync_copy(dis_hbm.at[pl.ds(base, RPT)], dis_v)
        for d in ds_:
            d.wait()

    @pl.when(sid == NS - 1)
    def _():
        ds_ = [
            pltpu.async_copy(acc3_hbm.at[0, pl.ds(base, LASTR)],
                             a0_v.at[pl.ds(0, LASTR)], ssems.at[0]),
            pltpu.async_copy(acc3_hbm.at[1, pl.ds(base, LASTR)],
                             a1_v.at[pl.ds(0, LASTR)], ssems.at[1]),
            pltpu.async_copy(t3_hbm.at[pl.ds(base, LASTR)],
                             t3_v.at[pl.ds(0, LASTR)], ssems.at[2]),
            pltpu.async_copy(h2_hbm.at[pl.ds(base, LASTR)],
                             h2_v.at[pl.ds(0, LASTR)], ssems.at[3]),
        ]
        pltpu.sync_copy(dis_hbm.at[pl.ds(base, LASTR)],
                        dis_v.at[pl.ds(0, LASTR)])
        for d in ds_:
            d.wait()

    pltpu.sync_copy(b3_hbm, b3_v)
    for d in stage:
        d.wait()
    b3v = b3_v[0:L]
    ngroups = jnp.where(sid == NS - 1, LASTR // L, RPT // L)

    def build(i, carry):
        dis16 = dis_v[pl.ds(i * L, L)]
        for j in range(L):
            r = i * L + j
            ds = jnp.take_along_axis(dis16, jnp.full((L,), j, jnp.int32),
                                     axis=0)
            conv = ds * (a0_v[r, :] + a1_v[r, :] + t3_v[r, :]) + b3v
            h3 = jnp.maximum(conv + h2_v[r, :], 0.0)
            t4_v[r, :] = ds * h3
        return carry

    lax.fori_loop(0, ngroups, build, 0)

    @pl.when(sid < NS - 1)
    def _():
        pltpu.sync_copy(t4_v, tbl_sh.at[pl.ds(base, RPT)])

    @pl.when(sid == NS - 1)
    def _():
        pltpu.sync_copy(t4_v.at[pl.ds(0, LASTR)],
                        tbl_sh.at[pl.ds(base, LASTR)])

    @pl.when(cid == 0)
    def _():
        pltpu.sync_copy(t4_v, t4_hbm.at[pl.ds(base, RPT)])

    plsc.subcore_barrier()
    _propagate(src_v, dst_v, w_v, rows_v, gsems, ssems, tbl_sh, acc_sh)
    plsc.subcore_barrier()
    pltpu.sync_copy(acc_sh.at[pl.ds(sid * RPT, RPT)],
                    out_hbm.at[cid, pl.ds(sid * RPT, RPT)])


BN = 1000


def _tc_a_body(deg_ref, x_ref, W1_ref, Wres_ref, bres_ref,
               t1_ref, xres_ref, dis_ref):
    deg = deg_ref[0] + deg_ref[1] + 1.0
    dis = lax.rsqrt(deg)
    xw = jnp.dot(x_ref[...], W1_ref[...], preferred_element_type=jnp.float32,
                precision=jax.lax.Precision.HIGHEST)
    t1_ref[...] = xw * dis
    xres_ref[...] = (
        jnp.dot(x_ref[...], Wres_ref[...], preferred_element_type=jnp.float32,
                precision=jax.lax.Precision.HIGHEST)
        + bres_ref[...]
    )
    dis_ref[...] = dis


_tc_a = pl.pallas_call(
    _tc_a_body,
    grid=(N // BN,),
    in_specs=[
        pl.BlockSpec((NC, BN, 1), lambda i: (0, i, 0)),
        pl.BlockSpec((BN, DIN), lambda i: (i, 0)),
        pl.BlockSpec((DIN, H), lambda i: (0, 0)),
        pl.BlockSpec((DIN, H), lambda i: (0, 0)),
        pl.BlockSpec((1, H), lambda i: (0, 0)),
    ],
    out_specs=[
        pl.BlockSpec((BN, H), lambda i: (i, 0)),
        pl.BlockSpec((BN, H), lambda i: (i, 0)),
        pl.BlockSpec((BN, 1), lambda i: (i, 0)),
    ],
    out_shape=[
        jax.ShapeDtypeStruct((N, H), jnp.float32),
        jax.ShapeDtypeStruct((N, H), jnp.float32),
        jax.ShapeDtypeStruct((N, 1), jnp.float32),
    ],
)


def _tc_b_body(acc_ref, t_ref, res_ref, dis_ref, b_ref, Wn_ref,
               h_ref, tn_ref):
    conv = (acc_ref[0] + acc_ref[1] + t_ref[...]) * dis_ref[...] + b_ref[...]
    h = jnp.maximum(conv + res_ref[...], 0.0)
    h_ref[...] = h
    tn_ref[...] = (
        jnp.dot(h, Wn_ref[...], preferred_element_type=jnp.float32,
                precision=jax.lax.Precision.HIGHEST)
        * dis_ref[...]
    )


_tc_b = pl.pallas_call(
    _tc_b_body,
    grid=(N // BN,),
    in_specs=[
        pl.BlockSpec((NC, BN, H), lambda i: (0, i, 0)),
        pl.BlockSpec((BN, H), lambda i: (i, 0)),
        pl.BlockSpec((BN, H), lambda i: (i, 0)),
        pl.BlockSpec((BN, 1), lambda i: (i, 0)),
        pl.BlockSpec((1, H), lambda i: (0, 0)),
        pl.BlockSpec((H, H), lambda i: (0, 0)),
    ],
    out_specs=[
        pl.BlockSpec((BN, H), lambda i: (i, 0)),
        pl.BlockSpec((BN, H), lambda i: (i, 0)),
    ],
    out_shape=[
        jax.ShapeDtypeStruct((N, H), jnp.float32),
        jax.ShapeDtypeStruct((N, H), jnp.float32),
    ],
)


def _tc_b3_body(acc_ref, t_ref, res_ref, dis_ref, b_ref, t4_ref):
    conv = (acc_ref[0] + acc_ref[1] + t_ref[...]) * dis_ref[...] + b_ref[...]
    h = jnp.maximum(conv + res_ref[...], 0.0)
    t4_ref[...] = h * dis_ref[...]


_tc_b3 = pl.pallas_call(
    _tc_b3_body,
    grid=(N // BN,),
    in_specs=[
        pl.BlockSpec((NC, BN, H), lambda i: (0, i, 0)),
        pl.BlockSpec((BN, H), lambda i: (i, 0)),
        pl.BlockSpec((BN, H), lambda i: (i, 0)),
        pl.BlockSpec((BN, 1), lambda i: (i, 0)),
        pl.BlockSpec((1, H), lambda i: (0, 0)),
    ],
    out_specs=[pl.BlockSpec((BN, H), lambda i: (i, 0))],
    out_shape=[jax.ShapeDtypeStruct((N, H), jnp.float32)],
)


def _tc_c_body(acc_ref, t_ref, dis_ref, W4_ref, b4_ref, out_ref):
    z = (acc_ref[0] + acc_ref[1] + t_ref[...]) * dis_ref[...]
    out_ref[...] = (
        jnp.dot(z, W4_ref[...], preferred_element_type=jnp.float32,
                precision=jax.lax.Precision.HIGHEST)
        + b4_ref[...]
    )


_tc_c = pl.pallas_call(
    _tc_c_body,
    grid=(N // BN,),
    in_specs=[
        pl.BlockSpec((NC, BN, H), lambda i: (0, i, 0)),
        pl.BlockSpec((BN, H), lambda i: (i, 0)),
        pl.BlockSpec((BN, 1), lambda i: (i, 0)),
        pl.BlockSpec((H, 1), lambda i: (0, 0)),
        pl.BlockSpec((1, 1), lambda i: (0, 0)),
    ],
    out_specs=[pl.BlockSpec((BN, 1), lambda i: (i, 0))],
    out_shape=[jax.ShapeDtypeStruct((N, 1), jnp.float32)],
)


def kernel(x, edge_index, edge_weight, W1, b1, W2, b2, W3, b3, W4, b4,
           Wres, bres):
    src = edge_index[0].astype(jnp.int32)
    dst = edge_index[1].astype(jnp.int32)
    w = edge_weight.astype(jnp.float32)
    e_in = src.shape[0]
    pad = E_CAP - e_in

    spread = (jnp.arange(pad, dtype=jnp.int32) * 97) % N
    src_p = jnp.concatenate([src, spread]).reshape(NW, NWIN, K)
    dummy = jnp.broadcast_to(
        ((jnp.arange(NBUF * K, dtype=jnp.int32) * 53) % N).reshape(1, NBUF, K),
        (NW, NBUF, K),
    )
    src3 = jnp.concatenate([src_p, dummy], axis=1)
    dst3 = jnp.concatenate([dst, spread]).reshape(NW, NWIN, K)
    w3 = jnp.concatenate([w, jnp.zeros((pad,), jnp.float32)]).reshape(
        NW, NWIN, K)
    zeros_n = jnp.zeros((NPAD,), jnp.float32)
    zeros_nh = jnp.zeros((NPAD, H), jnp.float32)

    _deg_kernel, _prop_kernel, _prop_epi_kernel = _sc_kernels()
    deg_parts = _deg_kernel(dst3, w3, zeros_n)
    t1, xres, dis = _tc_a(deg_parts.reshape(NC, NPAD, 1), x, W1, Wres,
                          bres.reshape(1, H))
    acc1 = _prop_kernel(t1, src3, dst3, w3, zeros_nh)
    h1, t2 = _tc_b(acc1, t1, xres, dis, b1.reshape(1, H), W2)
    acc2 = _prop_kernel(t2, src3, dst3, w3, zeros_nh)
    h2, t3 = _tc_b(acc2, t2, h1, dis, b2.reshape(1, H), W3)
    acc3 = _prop_kernel(t3, src3, dst3, w3, zeros_nh)
    acc4, t4 = _prop_epi_kernel(acc3, t3, h2, dis.reshape(N), b3,
                                src3, dst3, w3, zeros_nh)
    (out,) = _tc_c(acc4, t4[:N], dis, W4, b4.reshape(1, 1))
    return out


# speedup vs baseline: 1.2434x; 1.0673x over previous
import jax
import jax.numpy as jnp
from jax import lax
from jax.experimental import pallas as pl
from jax.experimental.pallas import tpu as pltpu
from jax.experimental.pallas import tpu_sc as plsc

N = 10000
DIN = 128
H = 16

NC = 2
NS = 16
L = 16
NW = NC * NS
K = 128
NWIN = 80
T_EDGES = K * NWIN
E_CAP = NW * T_EDGES
NPAD = 10240
RPT = NPAD // NS

def _deg_body(dst_hbm, w_hbm, zeros_hbm, out_hbm, dst_v, w_v, dsem, deg_sh):
    cid = lax.axis_index("c")
    sid = lax.axis_index("s")
    wid = sid * NC + cid
    d1 = pltpu.async_copy(dst_hbm.at[wid], dst_v, dsem)
    d2 = pltpu.async_copy(w_hbm.at[wid], w_v, dsem)
    pltpu.sync_copy(zeros_hbm.at[pl.ds(sid * RPT, RPT)],
                    deg_sh.at[pl.ds(sid * RPT, RPT)])
    d1.wait()
    d2.wait()
    plsc.subcore_barrier()

    def body(g, carry):
        pltpu.async_copy(w_v.at[g], deg_sh.at[dst_v.at[g]], dsem, add=True)

        @pl.when(g >= 8)
        def _():
            pltpu.make_async_copy(w_v.at[g], deg_sh.at[dst_v.at[g]],
                                  dsem).wait()
        return carry

    lax.fori_loop(0, NWIN, body, 0)
    for g in range(8):
        pltpu.make_async_copy(w_v.at[g], deg_sh.at[dst_v.at[g]], dsem).wait()
    plsc.subcore_barrier()
    pltpu.sync_copy(deg_sh.at[pl.ds(sid * RPT, RPT)],
                    out_hbm.at[cid, pl.ds(sid * RPT, RPT)])


import functools


@functools.cache
def _sc_kernels():
    mesh = plsc.VectorSubcoreMesh(
        core_axis_name="c", subcore_axis_name="s",
        num_cores=NC, num_subcores=NS,
    )
    deg_kernel = pl.kernel(
        _deg_body,
        out_type=jax.ShapeDtypeStruct((NC, NPAD), jnp.float32),
        mesh=mesh,
        scratch_types=[
            pltpu.VMEM((NWIN, K), jnp.int32),
            pltpu.VMEM((NWIN, K), jnp.float32),
            pltpu.SemaphoreType.DMA,
            pltpu.VMEM_SHARED((NPAD,), jnp.float32),
        ],
        compiler_params=pltpu.CompilerParams(use_tc_tiling_on_sc=False),
    )
    prop_scratch = [
        pltpu.VMEM((NWIN + NBUF, K), jnp.int32),
        pltpu.VMEM((NWIN, K), jnp.int32),
        pltpu.VMEM((NWIN, K), jnp.float32),
        pltpu.VMEM((NBUF, K, H), jnp.float32),
        pltpu.SemaphoreType.DMA((NBUF,)),
        pltpu.SemaphoreType.DMA((NBUF,)),
    ]
    prop_kernel = pl.kernel(
        _prop_body,
        out_type=jax.ShapeDtypeStruct((NC, NPAD, H), jnp.float32),
        mesh=mesh,
        scratch_types=prop_scratch + [
            pltpu.VMEM_SHARED((NPAD, H), jnp.float32),
            pltpu.VMEM_SHARED((NPAD, H), jnp.float32),
        ],
        compiler_params=pltpu.CompilerParams(use_tc_tiling_on_sc=False),
    )
    prop_epi_kernel = pl.kernel(
        _prop_epi_body,
        out_type=(jax.ShapeDtypeStruct((NC, NPAD, H), jnp.float32),
                  jax.ShapeDtypeStruct((NPAD, H), jnp.float32)),
        mesh=mesh,
        scratch_types=prop_scratch + [
            pltpu.VMEM((RPT, H), jnp.float32),
            pltpu.VMEM((RPT, H), jnp.float32),
            pltpu.VMEM((RPT, H), jnp.float32),
            pltpu.VMEM((RPT, H), jnp.float32),
            pltpu.VMEM((RPT,), jnp.float32),
            pltpu.VMEM((L,), jnp.float32),
            pltpu.VMEM((RPT, H), jnp.float32),
            pltpu.VMEM_SHARED((NPAD, H), jnp.float32),
            pltpu.VMEM_SHARED((NPAD, H), jnp.float32),
        ],
        compiler_params=pltpu.CompilerParams(use_tc_tiling_on_sc=False),
    )
    return deg_kernel, prop_kernel, prop_epi_kernel


NBUF = 4


LASTR = N - (NS - 1) * RPT


def _stage_edges_and_zero(src_hbm, dst_hbm, w_hbm, zeros_hbm,
                          src_v, dst_v, w_v, acc_sh, wid, sid, gsems):
    return [
        pltpu.async_copy(src_hbm.at[wid], src_v, gsems.at[0]),
        pltpu.async_copy(dst_hbm.at[wid], dst_v, gsems.at[1]),
        pltpu.async_copy(w_hbm.at[wid], w_v, gsems.at[2]),
        pltpu.async_copy(zeros_hbm.at[pl.ds(sid * RPT, RPT)],
                         acc_sh.at[pl.ds(sid * RPT, RPT)], gsems.at[3]),
    ]


def _propagate(src_v, dst_v, w_v, rows_v, gsems, ssems, tbl_sh, acc_sh):
    for b in range(NBUF):
        pltpu.async_copy(tbl_sh.at[src_v.at[b]], rows_v.at[b], gsems.at[b])

    def scale(g, b):
        rows = rows_v.at[b]
        for j16 in range(K // L):
            w16 = w_v[g, j16 * L:(j16 + 1) * L]
            for j in range(L):
                e = j16 * L + j
                ws = jnp.take_along_axis(
                    w16, jnp.full((L,), j, jnp.int32), axis=0)
                rows[e, :] = rows[e, :] * ws

    def body(g4, carry):
        for b in range(NBUF):
            g = g4 * NBUF + b
            pltpu.make_async_copy(
                tbl_sh.at[src_v.at[g]], rows_v.at[b], gsems.at[b]).wait()
            scale(g, b)
            pltpu.async_copy(rows_v.at[b], acc_sh.at[dst_v.at[g]],
                             ssems.at[b], add=True)
        for b in range(NBUF):
            g = g4 * NBUF + b
            pltpu.make_async_copy(rows_v.at[b], acc_sh.at[dst_v.at[g]],
                                  ssems.at[b]).wait()
            pltpu.async_copy(tbl_sh.at[src_v.at[g + NBUF]], rows_v.at[b],
                             gsems.at[b])
        return carry

    lax.fori_loop(0, NWIN // NBUF, body, 0)
    for b in range(NBUF):
        pltpu.make_async_copy(
            tbl_sh.at[src_v.at[b]], rows_v.at[b], gsems.at[b]).wait()


def _prop_body(t_hbm, src_hbm, dst_hbm, w_hbm, zeros_hbm, out_hbm,
               src_v, dst_v, w_v, rows_v, gsems, ssems, tbl_sh, acc_sh):
    cid = lax.axis_index("c")
    sid = lax.axis_index("s")
    wid = sid * NC + cid
    stage = _stage_edges_and_zero(src_hbm, dst_hbm, w_hbm, zeros_hbm,
                                  src_v, dst_v, w_v, acc_sh, wid, sid, gsems)

    @pl.when(sid < NS - 1)
    def _():
        pltpu.sync_copy(t_hbm.at[pl.ds(sid * RPT, RPT)],
                        tbl_sh.at[pl.ds(sid * RPT, RPT)])

    @pl.when(sid == NS - 1)
    def _():
        pltpu.sync_copy(t_hbm.at[pl.ds((NS - 1) * RPT, LASTR)],
                        tbl_sh.at[pl.ds((NS - 1) * RPT, LASTR)])

    for d in stage:
        d.wait()
    plsc.subcore_barrier()
    _propagate(src_v, dst_v, w_v, rows_v, gsems, ssems, tbl_sh, acc_sh)
    plsc.subcore_barrier()
    pltpu.sync_copy(acc_sh.at[pl.ds(sid * RPT, RPT)],
                    out_hbm.at[cid, pl.ds(sid * RPT, RPT)])


def _prop_epi_body(acc3_hbm, t3_hbm, h2_hbm, dis_hbm, b3_hbm,
                   src_hbm, dst_hbm, w_hbm, zeros_hbm,
                   out_hbm, t4_hbm,
                   src_v, dst_v, w_v, rows_v, gsems, ssems,
                   a0_v, a1_v, t3_v, h2_v, dis_v, b3_v, t4_v,
                   tbl_sh, acc_sh):
    cid = lax.axis_index("c")
    sid = lax.axis_index("s")
    wid = sid * NC + cid
    stage = _stage_edges_and_zero(src_hbm, dst_hbm, w_hbm, zeros_hbm,
                                  src_v, dst_v, w_v, acc_sh, wid, sid, gsems)

    base = sid * RPT

    @pl.when(sid < NS - 1)
    def _():
        ds_ = [
            pltpu.async_copy(acc3_hbm.at[0, pl.ds(base, RPT)], a0_v,
                             ssems.at[0]),
            pltpu.async_copy(acc3_hbm.at[1, pl.ds(base, RPT)], a1_v,
                             ssems.at[1]),
            pltpu.async_copy(t3_hbm.at[pl.ds(base, RPT)], t3_v,
                             ssems.at[2]),
            pltpu.async_copy(h2_hbm.at[pl.ds(base, RPT)], h2_v,
                             ssems.at[3]),
        ]
        pltpu.sync_copy(dis_hbm.at[pl.ds(base, RPT)], dis_v)
        for d in ds_:
            d.wait()

    @pl.when(sid == NS - 1)
    def _():
        ds_ = [
            pltpu.async_copy(acc3_hbm.at[0, pl.ds(base, LASTR)],
                             a0_v.at[pl.ds(0, LASTR)], ssems.at[0]),
            pltpu.async_copy(acc3_hbm.at[1, pl.ds(base, LASTR)],
                             a1_v.at[pl.ds(0, LASTR)], ssems.at[1]),
            pltpu.async_copy(t3_hbm.at[pl.ds(base, LASTR)],
                             t3_v.at[pl.ds(0, LASTR)], ssems.at[2]),
            pltpu.async_copy(h2_hbm.at[pl.ds(base, LASTR)],
                             h2_v.at[pl.ds(0, LASTR)], ssems.at[3]),
        ]
        pltpu.sync_copy(dis_hbm.at[pl.ds(base, LASTR)],
                        dis_v.at[pl.ds(0, LASTR)])
        for d in ds_:
            d.wait()

    pltpu.sync_copy(b3_hbm, b3_v)
    for d in stage:
        d.wait()
    b3v = b3_v[0:L]
    ngroups = jnp.where(sid == NS - 1, LASTR // L, RPT // L)

    def build(i, carry):
        dis16 = dis_v[pl.ds(i * L, L)]
        for j in range(L):
            r = i * L + j
            ds = jnp.take_along_axis(dis16, jnp.full((L,), j, jnp.int32),
                                     axis=0)
            conv = ds * (a0_v[r, :] + a1_v[r, :] + t3_v[r, :]) + b3v
            h3 = jnp.maximum(conv + h2_v[r, :], 0.0)
            t4_v[r, :] = ds * h3
        return carry

    lax.fori_loop(0, ngroups, build, 0)

    @pl.when(sid < NS - 1)
    def _():
        pltpu.sync_copy(t4_v, tbl_sh.at[pl.ds(base, RPT)])

    @pl.when(sid == NS - 1)
    def _():
        pltpu.sync_copy(t4_v.at[pl.ds(0, LASTR)],
                        tbl_sh.at[pl.ds(base, LASTR)])

    @pl.when(cid == 0)
    def _():
        pltpu.sync_copy(t4_v, t4_hbm.at[pl.ds(base, RPT)])

    plsc.subcore_barrier()
    _propagate(src_v, dst_v, w_v, rows_v, gsems, ssems, tbl_sh, acc_sh)
    plsc.subcore_barrier()
    pltpu.sync_copy(acc_sh.at[pl.ds(sid * RPT, RPT)],
                    out_hbm.at[cid, pl.ds(sid * RPT, RPT)])


BN = 2000


def _tc_a_body(deg_ref, x_ref, W1_ref, Wres_ref, bres_ref,
               t1_ref, xres_ref, dis_ref):
    deg = deg_ref[0] + deg_ref[1] + 1.0
    dis = lax.rsqrt(deg)
    xw = jnp.dot(x_ref[...], W1_ref[...], preferred_element_type=jnp.float32,
                precision=jax.lax.Precision.HIGHEST)
    t1_ref[...] = xw * dis
    xres_ref[...] = (
        jnp.dot(x_ref[...], Wres_ref[...], preferred_element_type=jnp.float32,
                precision=jax.lax.Precision.HIGHEST)
        + bres_ref[...]
    )
    dis_ref[...] = dis


_tc_a = pl.pallas_call(
    _tc_a_body,
    grid=(N // BN,),
    in_specs=[
        pl.BlockSpec((NC, BN, 1), lambda i: (0, i, 0)),
        pl.BlockSpec((BN, DIN), lambda i: (i, 0)),
        pl.BlockSpec((DIN, H), lambda i: (0, 0)),
        pl.BlockSpec((DIN, H), lambda i: (0, 0)),
        pl.BlockSpec((1, H), lambda i: (0, 0)),
    ],
    out_specs=[
        pl.BlockSpec((BN, H), lambda i: (i, 0)),
        pl.BlockSpec((BN, H), lambda i: (i, 0)),
        pl.BlockSpec((BN, 1), lambda i: (i, 0)),
    ],
    out_shape=[
        jax.ShapeDtypeStruct((N, H), jnp.float32),
        jax.ShapeDtypeStruct((N, H), jnp.float32),
        jax.ShapeDtypeStruct((N, 1), jnp.float32),
    ],
)


def _tc_b_body(acc_ref, t_ref, res_ref, dis_ref, b_ref, Wn_ref,
               h_ref, tn_ref):
    conv = (acc_ref[0] + acc_ref[1] + t_ref[...]) * dis_ref[...] + b_ref[...]
    h = jnp.maximum(conv + res_ref[...], 0.0)
    h_ref[...] = h
    tn_ref[...] = (
        jnp.dot(h, Wn_ref[...], preferred_element_type=jnp.float32,
                precision=jax.lax.Precision.HIGHEST)
        * dis_ref[...]
    )


_tc_b = pl.pallas_call(
    _tc_b_body,
    grid=(N // BN,),
    in_specs=[
        pl.BlockSpec((NC, BN, H), lambda i: (0, i, 0)),
        pl.BlockSpec((BN, H), lambda i: (i, 0)),
        pl.BlockSpec((BN, H), lambda i: (i, 0)),
        pl.BlockSpec((BN, 1), lambda i: (i, 0)),
        pl.BlockSpec((1, H), lambda i: (0, 0)),
        pl.BlockSpec((H, H), lambda i: (0, 0)),
    ],
    out_specs=[
        pl.BlockSpec((BN, H), lambda i: (i, 0)),
        pl.BlockSpec((BN, H), lambda i: (i, 0)),
    ],
    out_shape=[
        jax.ShapeDtypeStruct((N, H), jnp.float32),
        jax.ShapeDtypeStruct((N, H), jnp.float32),
    ],
)


def _tc_b3_body(acc_ref, t_ref, res_ref, dis_ref, b_ref, t4_ref):
    conv = (acc_ref[0] + acc_ref[1] + t_ref[...]) * dis_ref[...] + b_ref[...]
    h = jnp.maximum(conv + res_ref[...], 0.0)
    t4_ref[...] = h * dis_ref[...]


_tc_b3 = pl.pallas_call(
    _tc_b3_body,
    grid=(N // BN,),
    in_specs=[
        pl.BlockSpec((NC, BN, H), lambda i: (0, i, 0)),
        pl.BlockSpec((BN, H), lambda i: (i, 0)),
        pl.BlockSpec((BN, H), lambda i: (i, 0)),
        pl.BlockSpec((BN, 1), lambda i: (i, 0)),
        pl.BlockSpec((1, H), lambda i: (0, 0)),
    ],
    out_specs=[pl.BlockSpec((BN, H), lambda i: (i, 0))],
    out_shape=[jax.ShapeDtypeStruct((N, H), jnp.float32)],
)


def _tc_c_body(acc_ref, t_ref, dis_ref, W4_ref, b4_ref, out_ref):
    z = (acc_ref[0] + acc_ref[1] + t_ref[...]) * dis_ref[...]
    out_ref[...] = (
        jnp.dot(z, W4_ref[...], preferred_element_type=jnp.float32,
                precision=jax.lax.Precision.HIGHEST)
        + b4_ref[...]
    )


_tc_c = pl.pallas_call(
    _tc_c_body,
    grid=(N // BN,),
    in_specs=[
        pl.BlockSpec((NC, BN, H), lambda i: (0, i, 0)),
        pl.BlockSpec((BN, H), lambda i: (i, 0)),
        pl.BlockSpec((BN, 1), lambda i: (i, 0)),
        pl.BlockSpec((H, 1), lambda i: (0, 0)),
        pl.BlockSpec((1, 1), lambda i: (0, 0)),
    ],
    out_specs=[pl.BlockSpec((BN, 1), lambda i: (i, 0))],
    out_shape=[jax.ShapeDtypeStruct((N, 1), jnp.float32)],
)


def kernel(x, edge_index, edge_weight, W1, b1, W2, b2, W3, b3, W4, b4,
           Wres, bres):
    src = edge_index[0].astype(jnp.int32)
    dst = edge_index[1].astype(jnp.int32)
    w = edge_weight.astype(jnp.float32)
    e_in = src.shape[0]
    pad = E_CAP - e_in

    spread = (jnp.arange(pad, dtype=jnp.int32) * 97) % N
    src_p = jnp.concatenate([src, spread]).reshape(NW, NWIN, K)
    dummy = jnp.broadcast_to(
        ((jnp.arange(NBUF * K, dtype=jnp.int32) * 53) % N).reshape(1, NBUF, K),
        (NW, NBUF, K),
    )
    src3 = jnp.concatenate([src_p, dummy], axis=1)
    dst3 = jnp.concatenate([dst, spread]).reshape(NW, NWIN, K)
    w3 = jnp.concatenate([w, jnp.zeros((pad,), jnp.float32)]).reshape(
        NW, NWIN, K)
    zeros_n = jnp.zeros((NPAD,), jnp.float32)
    zeros_nh = jnp.zeros((NPAD, H), jnp.float32)

    _deg_kernel, _prop_kernel, _prop_epi_kernel = _sc_kernels()
    deg_parts = _deg_kernel(dst3, w3, zeros_n)
    t1, xres, dis = _tc_a(deg_parts.reshape(NC, NPAD, 1), x, W1, Wres,
                          bres.reshape(1, H))
    acc1 = _prop_kernel(t1, src3, dst3, w3, zeros_nh)
    h1, t2 = _tc_b(acc1, t1, xres, dis, b1.reshape(1, H), W2)
    acc2 = _prop_kernel(t2, src3, dst3, w3, zeros_nh)
    h2, t3 = _tc_b(acc2, t2, h1, dis, b2.reshape(1, H), W3)
    acc3 = _prop_kernel(t3, src3, dst3, w3, zeros_nh)
    acc4, t4 = _prop_epi_kernel(acc3, t3, h2, dis.reshape(N), b3,
                                src3, dst3, w3, zeros_nh)
    (out,) = _tc_c(acc4, t4[:N], dis, W4, b4.reshape(1, 1))
    return out
